# baseline (speedup 1.0000x reference)
import jax
import jax.numpy as jnp
from jax import lax
from jax.experimental import pallas as pl
from jax.experimental.pallas import tpu as pltpu

B = 16384
NUM_CLASSES = 101
OUT_DIM = 128
TPAD = 128
BLK = 4096
NB = B // BLK
SUB = BLK // 8


def _body(x_ref, w_ref, o_ref):
    xb = x_ref[0]
    iota = lax.broadcasted_iota(jnp.int32, (TPAD, SUB), 0)
    parts = [
        (xb[s:s + 1, :] == iota).astype(jnp.float32)
        for s in range(8)
    ]
    zt = jnp.concatenate(parts, axis=1)
    w = w_ref[...]
    sig = 1.0 / (1.0 + jnp.exp(-w))
    sig = jnp.concatenate(
        [sig, jnp.zeros((OUT_DIM, TPAD - NUM_CLASSES), jnp.float32)], axis=1
    )
    tbl = jnp.transpose(sig)
    o_ref[...] = lax.dot_general(
        zt, tbl,
        dimension_numbers=(((0,), (0,)), ((), ())),
        preferred_element_type=jnp.float32,
    )


@jax.jit
def _run(x_r, w):
    return pl.pallas_call(
        _body,
        grid=(NB,),
        in_specs=[
            pl.BlockSpec((1, 8, SUB), lambda i: (i, 0, 0)),
            pl.BlockSpec((OUT_DIM, NUM_CLASSES), lambda i: (0, 0)),
        ],
        out_specs=pl.BlockSpec((BLK, OUT_DIM), lambda i: (i, 0)),
        out_shape=jax.ShapeDtypeStruct((B, OUT_DIM), jnp.float32),
    )(x_r, w)


def kernel(x, W):
    return _run(x.reshape(NB, 8, SUB), W)


# speedup vs baseline: 1.2134x; 1.2134x over previous
import jax
import jax.numpy as jnp
from jax import lax
from jax.experimental import pallas as pl
from jax.experimental.pallas import tpu as pltpu

B = 16384
NUM_CLASSES = 101
OUT_DIM = 128
TPAD = 128
BLK = 4096
NB = B // BLK
SUB = 512


def _body(x_ref, w_ref, o_ref):
    iota = lax.broadcasted_iota(jnp.int32, (TPAD, SUB), 0)
    parts = [
        (x_ref[pl.ds(s * SUB, SUB)][None, :] == iota).astype(jnp.float32)
        for s in range(BLK // SUB)
    ]
    zt = jnp.concatenate(parts, axis=1)
    w = w_ref[...]
    sig = 1.0 / (1.0 + jnp.exp(-w))
    sig = jnp.concatenate(
        [sig, jnp.zeros((OUT_DIM, TPAD - NUM_CLASSES), jnp.float32)], axis=1
    )
    tbl = jnp.transpose(sig)
    o_ref[...] = lax.dot_general(
        zt, tbl,
        dimension_numbers=(((0,), (0,)), ((), ())),
        preferred_element_type=jnp.float32,
    )


@jax.jit
def _run(x, w):
    return pl.pallas_call(
        _body,
        grid=(NB,),
        in_specs=[
            pl.BlockSpec((BLK,), lambda i: (i,)),
            pl.BlockSpec((OUT_DIM, NUM_CLASSES), lambda i: (0, 0)),
        ],
        out_specs=pl.BlockSpec((BLK, OUT_DIM), lambda i: (i, 0)),
        out_shape=jax.ShapeDtypeStruct((B, OUT_DIM), jnp.float32),
    )(x, w)


def kernel(x, W):
    return _run(x, W)
